# Initial kernel scaffold; baseline (speedup 1.0000x reference)
import jax
import jax.numpy as jnp
from jax.experimental import pallas as pl


def kernel(x, edge_index, W1_rel, b1, W1_root, W2_rel, b2, W2_root):
    raise NotImplementedError("write your pallas kernel here")


# speedup vs baseline: 10.1711x; 10.1711x over previous
import functools

import jax
import jax.numpy as jnp
from jax import lax
from jax.experimental import pallas as pl
from jax.experimental.pallas import tpu as pltpu
from jax.experimental.pallas import tpu_sc as plsc

_NW = 32
_CHUNK = 128
_NSUB = 16


def _tc_dual_matmul(x, Wa, Wb):
    n, d = x.shape
    f = Wa.shape[1]
    bn = 1000
    assert n % bn == 0

    def body(x_ref, wa_ref, wb_ref, ya_ref, yb_ref):
        xv = x_ref[...]
        ya_ref[...] = jnp.dot(xv, wa_ref[...], preferred_element_type=jnp.float32)
        yb_ref[...] = jnp.dot(xv, wb_ref[...], preferred_element_type=jnp.float32)

    return pl.pallas_call(
        body,
        grid=(n // bn,),
        in_specs=[
            pl.BlockSpec((bn, d), lambda i: (i, 0)),
            pl.BlockSpec((d, f), lambda i: (0, 0)),
            pl.BlockSpec((d, f), lambda i: (0, 0)),
        ],
        out_specs=[
            pl.BlockSpec((bn, f), lambda i: (i, 0)),
            pl.BlockSpec((bn, f), lambda i: (i, 0)),
        ],
        out_shape=[
            jax.ShapeDtypeStruct((n, f), jnp.float32),
            jax.ShapeDtypeStruct((n, f), jnp.float32),
        ],
    )(x, Wa, Wb)


def _segment_sum_sc(y, src2d, dst2d, zeros, nacc):
    n_chunks = src2d.shape[0]
    per_w = n_chunks // _NW
    f = y.shape[1]
    rows_per_sub = nacc // _NSUB
    mesh = plsc.VectorSubcoreMesh(core_axis_name="c", subcore_axis_name="s")

    @functools.partial(
        pl.kernel,
        mesh=mesh,
        compiler_params=pltpu.CompilerParams(use_tc_tiling_on_sc=False),
        out_type=jax.ShapeDtypeStruct((2, nacc, f), jnp.float32),
        scratch_types=[
            pltpu.VMEM((per_w, _CHUNK), jnp.int32),
            pltpu.VMEM((per_w, _CHUNK), jnp.int32),
            pltpu.VMEM((_CHUNK, f), jnp.float32),
            pltpu.VMEM_SHARED((nacc, f), jnp.float32),
            pltpu.SemaphoreType.DMA,
        ],
    )
    def k(y_hbm, src_hbm, dst_hbm, zero_hbm, out_hbm, src_v, dst_v, rows_v,
          acc_sh, sem):
        cid = lax.axis_index("c")
        sid = lax.axis_index("s")
        wid = sid * 2 + cid
        r0 = sid * rows_per_sub
        pltpu.sync_copy(zero_hbm.at[pl.ds(r0, rows_per_sub)],
                        acc_sh.at[pl.ds(r0, rows_per_sub)])
        pltpu.sync_copy(src_hbm.at[pl.ds(wid * per_w, per_w)], src_v)
        pltpu.sync_copy(dst_hbm.at[pl.ds(wid * per_w, per_w)], dst_v)
        plsc.subcore_barrier()

        def body(i, carry):
            pltpu.async_copy(y_hbm.at[src_v.at[i]], rows_v, sem).wait()
            pltpu.sync_copy(rows_v, acc_sh.at[dst_v.at[i]], add=True)
            return carry

        lax.fori_loop(0, per_w, body, 0)
        plsc.subcore_barrier()
        pltpu.sync_copy(acc_sh.at[pl.ds(r0, rows_per_sub)],
                        out_hbm.at[cid, pl.ds(r0, rows_per_sub)])

    return k(y, src2d, dst2d, zeros)


def _tc_mid(agg, r1, b1, W2rel_p, W2root_p, b2_p, n):
    f = r1.shape[1]
    bn = 1000
    nacc = agg.shape[1]

    def body(agg_ref, r1_ref, b1_ref, wrel_ref, wroot_ref, b2_ref,
             y2_ref, r2_ref):
        h = jnp.maximum(
            agg_ref[0] + agg_ref[1] + b1_ref[...] + r1_ref[...], 0.0)
        y2_ref[...] = jnp.dot(h, wrel_ref[...],
                              preferred_element_type=jnp.float32)
        r2_ref[...] = jnp.dot(h, wroot_ref[...],
                              preferred_element_type=jnp.float32) + b2_ref[...]

    return pl.pallas_call(
        body,
        grid=(n // bn,),
        in_specs=[
            pl.BlockSpec((2, bn, f), lambda i: (0, i, 0)),
            pl.BlockSpec((bn, f), lambda i: (i, 0)),
            pl.BlockSpec((1, f), lambda i: (0, 0)),
            pl.BlockSpec((f, f), lambda i: (0, 0)),
            pl.BlockSpec((f, f), lambda i: (0, 0)),
            pl.BlockSpec((1, f), lambda i: (0, 0)),
        ],
        out_specs=[
            pl.BlockSpec((bn, f), lambda i: (i, 0)),
            pl.BlockSpec((bn, f), lambda i: (i, 0)),
        ],
        out_shape=[
            jax.ShapeDtypeStruct((n, f), jnp.float32),
            jax.ShapeDtypeStruct((n, f), jnp.float32),
        ],
    )(agg, r1, b1, W2rel_p, W2root_p, b2_p)


def _tc_logsoftmax(agg2, r2, n, c):
    f = r2.shape[1]
    bn = 1000

    def body(agg_ref, r2_ref, o_ref):
        logits = agg_ref[0] + agg_ref[1] + r2_ref[...]
        col = lax.broadcasted_iota(jnp.int32, logits.shape, 1)
        valid = col < c
        masked = jnp.where(valid, logits, -jnp.inf)
        m = jnp.max(masked, axis=1, keepdims=True)
        ex = jnp.where(valid, jnp.exp(logits - m), 0.0)
        s = jnp.sum(ex, axis=1, keepdims=True)
        o_ref[...] = logits - m - jnp.log(s)

    return pl.pallas_call(
        body,
        grid=(n // bn,),
        in_specs=[
            pl.BlockSpec((2, bn, f), lambda i: (0, i, 0)),
            pl.BlockSpec((bn, f), lambda i: (i, 0)),
        ],
        out_specs=pl.BlockSpec((bn, f), lambda i: (i, 0)),
        out_shape=jax.ShapeDtypeStruct((n, f), jnp.float32),
    )(agg2, r2)


def kernel(x, edge_index, W1_rel, b1, W1_root, W2_rel, b2, W2_root):
    n, d = x.shape
    e = edge_index.shape[1]
    h = W1_rel.shape[1]
    c = W2_rel.shape[1]
    nacc = -(-(n + 1) // (_NSUB * 8)) * (_NSUB * 8)

    n_chunks = -(-e // (_CHUNK * _NW * 8)) * (_NW * 8)
    pad = n_chunks * _CHUNK - e
    src = jnp.concatenate([edge_index[0], jnp.zeros((pad,), jnp.int32)])
    dst = jnp.concatenate([edge_index[1], jnp.full((pad,), n, jnp.int32)])
    src2d = src.reshape(n_chunks, _CHUNK)
    dst2d = dst.reshape(n_chunks, _CHUNK)
    zeros = jnp.zeros((nacc, h), jnp.float32)

    W2rel_p = jnp.zeros((h, h), jnp.float32).at[:, :c].set(W2_rel)
    W2root_p = jnp.zeros((h, h), jnp.float32).at[:, :c].set(W2_root)
    b1_2d = b1.reshape(1, h)
    b2_p = jnp.zeros((1, h), jnp.float32).at[0, :c].set(b2)

    y1, r1 = _tc_dual_matmul(x, W1_rel, W1_root)
    agg1 = _segment_sum_sc(y1, src2d, dst2d, zeros, nacc)
    y2, r2 = _tc_mid(agg1, r1, b1_2d, W2rel_p, W2root_p, b2_p, n)
    agg2 = _segment_sum_sc(y2, src2d, dst2d, zeros, nacc)
    out = _tc_logsoftmax(agg2, r2, n, c)
    return out[:, :c]


# speedup vs baseline: 13.8767x; 1.3643x over previous
import functools

import jax
import jax.numpy as jnp
from jax import lax
from jax.experimental import pallas as pl
from jax.experimental.pallas import tpu as pltpu
from jax.experimental.pallas import tpu_sc as plsc

_NW = 32
_CHUNK = 128
_NSUB = 16


def _tc_dual_matmul(x, Wa, Wb):
    n, d = x.shape
    f = Wa.shape[1]
    bn = 1000
    assert n % bn == 0

    def body(x_ref, wa_ref, wb_ref, ya_ref, yb_ref):
        xv = x_ref[...]
        ya_ref[...] = jnp.dot(xv, wa_ref[...], preferred_element_type=jnp.float32)
        yb_ref[...] = jnp.dot(xv, wb_ref[...], preferred_element_type=jnp.float32)

    return pl.pallas_call(
        body,
        grid=(n // bn,),
        in_specs=[
            pl.BlockSpec((bn, d), lambda i: (i, 0)),
            pl.BlockSpec((d, f), lambda i: (0, 0)),
            pl.BlockSpec((d, f), lambda i: (0, 0)),
        ],
        out_specs=[
            pl.BlockSpec((bn, f), lambda i: (i, 0)),
            pl.BlockSpec((bn, f), lambda i: (i, 0)),
        ],
        out_shape=[
            jax.ShapeDtypeStruct((n, f), jnp.float32),
            jax.ShapeDtypeStruct((n, f), jnp.float32),
        ],
    )(x, Wa, Wb)


def _segment_sum_sc(y, src2d, dst2d, zeros, nacc):
    n_chunks = src2d.shape[0]
    per_w = n_chunks // _NW
    f = y.shape[1]
    rows_per_sub = nacc // _NSUB
    nbuf = 4
    look = nbuf // 2
    ngroups = per_w // nbuf
    mesh = plsc.VectorSubcoreMesh(core_axis_name="c", subcore_axis_name="s")

    @functools.partial(
        pl.kernel,
        mesh=mesh,
        compiler_params=pltpu.CompilerParams(use_tc_tiling_on_sc=False),
        out_type=jax.ShapeDtypeStruct((2, nacc, f), jnp.float32),
        scratch_types=[
            pltpu.VMEM((per_w, _CHUNK), jnp.int32),
            pltpu.VMEM((per_w, _CHUNK), jnp.int32),
            pltpu.VMEM_SHARED((nacc, f), jnp.float32),
        ]
        + [pltpu.VMEM((_CHUNK, f), jnp.float32)] * nbuf
        + [pltpu.SemaphoreType.DMA] * (2 * nbuf),
    )
    def k(y_hbm, src_hbm, dst_hbm, zero_hbm, out_hbm, src_v, dst_v, acc_sh,
          *rest):
        rows = rest[:nbuf]
        gsem = rest[nbuf:2 * nbuf]
        ssem = rest[2 * nbuf:]
        cid = lax.axis_index("c")
        sid = lax.axis_index("s")
        wid = sid * 2 + cid
        r0 = sid * rows_per_sub
        pltpu.sync_copy(zero_hbm.at[pl.ds(r0, rows_per_sub)],
                        acc_sh.at[pl.ds(r0, rows_per_sub)])
        pltpu.sync_copy(src_hbm.at[pl.ds(wid * per_w, per_w)], src_v)
        pltpu.sync_copy(dst_hbm.at[pl.ds(wid * per_w, per_w)], dst_v)
        plsc.subcore_barrier()

        def gather_start(ki, b):
            pltpu.async_copy(y_hbm.at[src_v.at[ki]], rows[b], gsem[b])

        def gather_wait(ki, b):
            pltpu.make_async_copy(y_hbm.at[src_v.at[ki]], rows[b],
                                  gsem[b]).wait()

        def scat_start(ki, b):
            pltpu.async_copy(rows[b], acc_sh.at[dst_v.at[ki]], ssem[b],
                             add=True)

        def scat_wait(ki, b):
            pltpu.make_async_copy(rows[b], acc_sh.at[dst_v.at[ki]],
                                  ssem[b]).wait()

        for b in range(look):
            gather_start(b, b)
        for b in range(nbuf):
            gather_wait(b, b)
            scat_start(b, b)
            b2 = (b + look) % nbuf
            if b >= look:
                scat_wait(b - look, b2)
            gather_start(look + b, b2)

        def body(g, carry):
            for b in range(nbuf):
                ki = g * nbuf + b
                gather_wait(ki, b)
                scat_start(ki, b)
                b2 = (b + look) % nbuf
                scat_wait(ki - look, b2)

                @pl.when(ki + look < per_w)
                def _():
                    gather_start(ki + look, b2)
            return carry

        lax.fori_loop(1, ngroups, body, 0)
        for b in range(look, nbuf):
            scat_wait(per_w - nbuf + b, b)
        plsc.subcore_barrier()
        pltpu.sync_copy(acc_sh.at[pl.ds(r0, rows_per_sub)],
                        out_hbm.at[cid, pl.ds(r0, rows_per_sub)])

    return k(y, src2d, dst2d, zeros)


def _tc_mid(agg, r1, b1, W2rel_p, W2root_p, b2_p, n):
    f = r1.shape[1]
    bn = 1000
    nacc = agg.shape[1]

    def body(agg_ref, r1_ref, b1_ref, wrel_ref, wroot_ref, b2_ref,
             y2_ref, r2_ref):
        h = jnp.maximum(
            agg_ref[0] + agg_ref[1] + b1_ref[...] + r1_ref[...], 0.0)
        y2_ref[...] = jnp.dot(h, wrel_ref[...],
                              preferred_element_type=jnp.float32)
        r2_ref[...] = jnp.dot(h, wroot_ref[...],
                              preferred_element_type=jnp.float32) + b2_ref[...]

    return pl.pallas_call(
        body,
        grid=(n // bn,),
        in_specs=[
            pl.BlockSpec((2, bn, f), lambda i: (0, i, 0)),
            pl.BlockSpec((bn, f), lambda i: (i, 0)),
            pl.BlockSpec((1, f), lambda i: (0, 0)),
            pl.BlockSpec((f, f), lambda i: (0, 0)),
            pl.BlockSpec((f, f), lambda i: (0, 0)),
            pl.BlockSpec((1, f), lambda i: (0, 0)),
        ],
        out_specs=[
            pl.BlockSpec((bn, f), lambda i: (i, 0)),
            pl.BlockSpec((bn, f), lambda i: (i, 0)),
        ],
        out_shape=[
            jax.ShapeDtypeStruct((n, f), jnp.float32),
            jax.ShapeDtypeStruct((n, f), jnp.float32),
        ],
    )(agg, r1, b1, W2rel_p, W2root_p, b2_p)


def _tc_logsoftmax(agg2, r2, n, c):
    f = r2.shape[1]
    bn = 1000

    def body(agg_ref, r2_ref, o_ref):
        logits = agg_ref[0] + agg_ref[1] + r2_ref[...]
        col = lax.broadcasted_iota(jnp.int32, logits.shape, 1)
        valid = col < c
        masked = jnp.where(valid, logits, -jnp.inf)
        m = jnp.max(masked, axis=1, keepdims=True)
        ex = jnp.where(valid, jnp.exp(logits - m), 0.0)
        s = jnp.sum(ex, axis=1, keepdims=True)
        o_ref[...] = logits - m - jnp.log(s)

    return pl.pallas_call(
        body,
        grid=(n // bn,),
        in_specs=[
            pl.BlockSpec((2, bn, f), lambda i: (0, i, 0)),
            pl.BlockSpec((bn, f), lambda i: (i, 0)),
        ],
        out_specs=pl.BlockSpec((bn, f), lambda i: (i, 0)),
        out_shape=jax.ShapeDtypeStruct((n, f), jnp.float32),
    )(agg2, r2)


def kernel(x, edge_index, W1_rel, b1, W1_root, W2_rel, b2, W2_root):
    n, d = x.shape
    e = edge_index.shape[1]
    h = W1_rel.shape[1]
    c = W2_rel.shape[1]
    nacc = -(-(n + 1) // (_NSUB * 8)) * (_NSUB * 8)

    n_chunks = -(-e // (_CHUNK * _NW * 16)) * (_NW * 16)
    pad = n_chunks * _CHUNK - e
    src = jnp.concatenate([edge_index[0], jnp.zeros((pad,), jnp.int32)])
    dst = jnp.concatenate([edge_index[1], jnp.full((pad,), n, jnp.int32)])
    src2d = src.reshape(n_chunks, _CHUNK)
    dst2d = dst.reshape(n_chunks, _CHUNK)
    zeros = jnp.zeros((nacc, h), jnp.float32)

    W2rel_p = jnp.zeros((h, h), jnp.float32).at[:, :c].set(W2_rel)
    W2root_p = jnp.zeros((h, h), jnp.float32).at[:, :c].set(W2_root)
    b1_2d = b1.reshape(1, h)
    b2_p = jnp.zeros((1, h), jnp.float32).at[0, :c].set(b2)

    y1, r1 = _tc_dual_matmul(x, W1_rel, W1_root)
    agg1 = _segment_sum_sc(y1, src2d, dst2d, zeros, nacc)
    y2, r2 = _tc_mid(agg1, r1, b1_2d, W2rel_p, W2root_p, b2_p, n)
    agg2 = _segment_sum_sc(y2, src2d, dst2d, zeros, nacc)
    out = _tc_logsoftmax(agg2, r2, n, c)
    return out[:, :c]


# speedup vs baseline: 14.0509x; 1.0126x over previous
import functools

import jax
import jax.numpy as jnp
from jax import lax
from jax.experimental import pallas as pl
from jax.experimental.pallas import tpu as pltpu
from jax.experimental.pallas import tpu_sc as plsc

_NW = 32
_CHUNK = 128
_NSUB = 16


def _tc_dual_matmul(x, Wa, Wb):
    n, d = x.shape
    f = Wa.shape[1]
    bn = 1000
    assert n % bn == 0

    def body(x_ref, wa_ref, wb_ref, ya_ref, yb_ref):
        xv = x_ref[...]
        ya_ref[...] = jnp.dot(xv, wa_ref[...], preferred_element_type=jnp.float32)
        yb_ref[...] = jnp.dot(xv, wb_ref[...], preferred_element_type=jnp.float32)

    return pl.pallas_call(
        body,
        grid=(n // bn,),
        in_specs=[
            pl.BlockSpec((bn, d), lambda i: (i, 0)),
            pl.BlockSpec((d, f), lambda i: (0, 0)),
            pl.BlockSpec((d, f), lambda i: (0, 0)),
        ],
        out_specs=[
            pl.BlockSpec((bn, f), lambda i: (i, 0)),
            pl.BlockSpec((bn, f), lambda i: (i, 0)),
        ],
        out_shape=[
            jax.ShapeDtypeStruct((n, f), jnp.float32),
            jax.ShapeDtypeStruct((n, f), jnp.float32),
        ],
    )(x, Wa, Wb)


def _segment_sum_sc(y, src2d, dst2d, zeros, nacc):
    n_chunks = src2d.shape[0]
    per_w = n_chunks // _NW
    f = y.shape[1]
    rows_per_sub = nacc // _NSUB
    nbuf = 8
    look = nbuf // 2
    ngroups = per_w // nbuf
    mesh = plsc.VectorSubcoreMesh(core_axis_name="c", subcore_axis_name="s")

    @functools.partial(
        pl.kernel,
        mesh=mesh,
        compiler_params=pltpu.CompilerParams(use_tc_tiling_on_sc=False),
        out_type=jax.ShapeDtypeStruct((2, nacc, f), jnp.float32),
        scratch_types=[
            pltpu.VMEM((per_w, _CHUNK), jnp.int32),
            pltpu.VMEM((per_w, _CHUNK), jnp.int32),
            pltpu.VMEM_SHARED((nacc, f), jnp.float32),
        ]
        + [pltpu.VMEM((_CHUNK, f), jnp.float32)] * nbuf
        + [pltpu.SemaphoreType.DMA] * (2 * nbuf),
    )
    def k(y_hbm, src_hbm, dst_hbm, zero_hbm, out_hbm, src_v, dst_v, acc_sh,
          *rest):
        rows = rest[:nbuf]
        gsem = rest[nbuf:2 * nbuf]
        ssem = rest[2 * nbuf:]
        cid = lax.axis_index("c")
        sid = lax.axis_index("s")
        wid = sid * 2 + cid
        r0 = sid * rows_per_sub
        pltpu.sync_copy(zero_hbm.at[pl.ds(r0, rows_per_sub)],
                        acc_sh.at[pl.ds(r0, rows_per_sub)])
        pltpu.sync_copy(src_hbm.at[pl.ds(wid * per_w, per_w)], src_v)
        pltpu.sync_copy(dst_hbm.at[pl.ds(wid * per_w, per_w)], dst_v)
        plsc.subcore_barrier()

        def gather_start(ki, b):
            pltpu.async_copy(y_hbm.at[src_v.at[ki]], rows[b], gsem[b])

        def gather_wait(ki, b):
            pltpu.make_async_copy(y_hbm.at[src_v.at[ki]], rows[b],
                                  gsem[b]).wait()

        def scat_start(ki, b):
            pltpu.async_copy(rows[b], acc_sh.at[dst_v.at[ki]], ssem[b],
                             add=True)

        def scat_wait(ki, b):
            pltpu.make_async_copy(rows[b], acc_sh.at[dst_v.at[ki]],
                                  ssem[b]).wait()

        for b in range(look):
            gather_start(b, b)
        for b in range(nbuf):
            gather_wait(b, b)
            scat_start(b, b)
            b2 = (b + look) % nbuf
            if b >= look:
                scat_wait(b - look, b2)
            gather_start(look + b, b2)

        def body(g, carry):
            for b in range(nbuf):
                ki = g * nbuf + b
                gather_wait(ki, b)
                scat_start(ki, b)
                b2 = (b + look) % nbuf
                scat_wait(ki - look, b2)

                @pl.when(ki + look < per_w)
                def _():
                    gather_start(ki + look, b2)
            return carry

        lax.fori_loop(1, ngroups, body, 0)
        for b in range(look, nbuf):
            scat_wait(per_w - nbuf + b, b)
        plsc.subcore_barrier()
        pltpu.sync_copy(acc_sh.at[pl.ds(r0, rows_per_sub)],
                        out_hbm.at[cid, pl.ds(r0, rows_per_sub)])

    return k(y, src2d, dst2d, zeros)


def _tc_mid(agg, r1, b1, W2rel_p, W2root_p, b2_p, n):
    f = r1.shape[1]
    bn = 1000
    nacc = agg.shape[1]

    def body(agg_ref, r1_ref, b1_ref, wrel_ref, wroot_ref, b2_ref,
             y2_ref, r2_ref):
        h = jnp.maximum(
            agg_ref[0] + agg_ref[1] + b1_ref[...] + r1_ref[...], 0.0)
        y2_ref[...] = jnp.dot(h, wrel_ref[...],
                              preferred_element_type=jnp.float32)
        r2_ref[...] = jnp.dot(h, wroot_ref[...],
                              preferred_element_type=jnp.float32) + b2_ref[...]

    return pl.pallas_call(
        body,
        grid=(n // bn,),
        in_specs=[
            pl.BlockSpec((2, bn, f), lambda i: (0, i, 0)),
            pl.BlockSpec((bn, f), lambda i: (i, 0)),
            pl.BlockSpec((1, f), lambda i: (0, 0)),
            pl.BlockSpec((f, f), lambda i: (0, 0)),
            pl.BlockSpec((f, f), lambda i: (0, 0)),
            pl.BlockSpec((1, f), lambda i: (0, 0)),
        ],
        out_specs=[
            pl.BlockSpec((bn, f), lambda i: (i, 0)),
            pl.BlockSpec((bn, f), lambda i: (i, 0)),
        ],
        out_shape=[
            jax.ShapeDtypeStruct((n, f), jnp.float32),
            jax.ShapeDtypeStruct((n, f), jnp.float32),
        ],
    )(agg, r1, b1, W2rel_p, W2root_p, b2_p)


def _tc_logsoftmax(agg2, r2, n, c):
    f = r2.shape[1]
    bn = 1000

    def body(agg_ref, r2_ref, o_ref):
        logits = agg_ref[0] + agg_ref[1] + r2_ref[...]
        col = lax.broadcasted_iota(jnp.int32, logits.shape, 1)
        valid = col < c
        masked = jnp.where(valid, logits, -jnp.inf)
        m = jnp.max(masked, axis=1, keepdims=True)
        ex = jnp.where(valid, jnp.exp(logits - m), 0.0)
        s = jnp.sum(ex, axis=1, keepdims=True)
        o_ref[...] = logits - m - jnp.log(s)

    return pl.pallas_call(
        body,
        grid=(n // bn,),
        in_specs=[
            pl.BlockSpec((2, bn, f), lambda i: (0, i, 0)),
            pl.BlockSpec((bn, f), lambda i: (i, 0)),
        ],
        out_specs=pl.BlockSpec((bn, f), lambda i: (i, 0)),
        out_shape=jax.ShapeDtypeStruct((n, f), jnp.float32),
    )(agg2, r2)


def kernel(x, edge_index, W1_rel, b1, W1_root, W2_rel, b2, W2_root):
    n, d = x.shape
    e = edge_index.shape[1]
    h = W1_rel.shape[1]
    c = W2_rel.shape[1]
    nacc = -(-(n + 1) // (_NSUB * 8)) * (_NSUB * 8)

    n_chunks = -(-e // (_CHUNK * _NW * 16)) * (_NW * 16)
    pad = n_chunks * _CHUNK - e
    src = jnp.concatenate([edge_index[0], jnp.zeros((pad,), jnp.int32)])
    dst = jnp.concatenate([edge_index[1], jnp.full((pad,), n, jnp.int32)])
    src2d = src.reshape(n_chunks, _CHUNK)
    dst2d = dst.reshape(n_chunks, _CHUNK)
    zeros = jnp.zeros((nacc, h), jnp.float32)

    W2rel_p = jnp.zeros((h, h), jnp.float32).at[:, :c].set(W2_rel)
    W2root_p = jnp.zeros((h, h), jnp.float32).at[:, :c].set(W2_root)
    b1_2d = b1.reshape(1, h)
    b2_p = jnp.zeros((1, h), jnp.float32).at[0, :c].set(b2)

    y1, r1 = _tc_dual_matmul(x, W1_rel, W1_root)
    agg1 = _segment_sum_sc(y1, src2d, dst2d, zeros, nacc)
    y2, r2 = _tc_mid(agg1, r1, b1_2d, W2rel_p, W2root_p, b2_p, n)
    agg2 = _segment_sum_sc(y2, src2d, dst2d, zeros, nacc)
    out = _tc_logsoftmax(agg2, r2, n, c)
    return out[:, :c]


# speedup vs baseline: 18.1901x; 1.2946x over previous
import functools

import jax
import jax.numpy as jnp
from jax import lax
from jax.experimental import pallas as pl
from jax.experimental.pallas import tpu as pltpu
from jax.experimental.pallas import tpu_sc as plsc

_NW = 32
_CHUNK = 125
_NSUB = 16


def _tc_dual_matmul(x, Wa, Wb):
    n, d = x.shape
    f = Wa.shape[1]
    bn = 1000
    assert n % bn == 0

    def body(x_ref, wa_ref, wb_ref, ya_ref, yb_ref):
        xv = x_ref[...]
        ya_ref[...] = jnp.dot(xv, wa_ref[...], preferred_element_type=jnp.float32)
        yb_ref[...] = jnp.dot(xv, wb_ref[...], preferred_element_type=jnp.float32)

    return pl.pallas_call(
        body,
        grid=(n // bn,),
        in_specs=[
            pl.BlockSpec((bn, d), lambda i: (i, 0)),
            pl.BlockSpec((d, f), lambda i: (0, 0)),
            pl.BlockSpec((d, f), lambda i: (0, 0)),
        ],
        out_specs=[
            pl.BlockSpec((bn, f), lambda i: (i, 0)),
            pl.BlockSpec((bn, f), lambda i: (i, 0)),
        ],
        out_shape=[
            jax.ShapeDtypeStruct((n, f), jnp.float32),
            jax.ShapeDtypeStruct((n, f), jnp.float32),
        ],
    )(x, Wa, Wb)


def _segment_sum_sc(y, src2d, dst2d, zeros, nacc, split0):
    n_chunks = src2d.shape[0]
    f = y.shape[1]
    rows_per_sub = nacc // _NSUB
    nbuf = 8
    look = nbuf // 2
    pw0 = split0 // _NSUB
    pw1 = (n_chunks - split0) // _NSUB
    pwmax = max(pw0, pw1)
    ngroups = pwmax // nbuf
    mesh = plsc.VectorSubcoreMesh(core_axis_name="c", subcore_axis_name="s")

    @functools.partial(
        pl.kernel,
        mesh=mesh,
        compiler_params=pltpu.CompilerParams(use_tc_tiling_on_sc=False),
        out_type=jax.ShapeDtypeStruct((2, nacc, f), jnp.float32),
        scratch_types=[
            pltpu.VMEM((pwmax, _CHUNK), jnp.int32),
            pltpu.VMEM((pwmax, _CHUNK), jnp.int32),
            pltpu.VMEM_SHARED((nacc, f), jnp.float32),
        ]
        + [pltpu.VMEM((_CHUNK, f), jnp.float32)] * nbuf
        + [pltpu.SemaphoreType.DMA] * (2 * nbuf),
    )
    def k(y_hbm, src_hbm, dst_hbm, zero_hbm, out_hbm, src_v, dst_v, acc_sh,
          *rest):
        rows = rest[:nbuf]
        gsem = rest[nbuf:2 * nbuf]
        ssem = rest[2 * nbuf:]
        cid = lax.axis_index("c")
        sid = lax.axis_index("s")
        r0 = sid * rows_per_sub
        pe = jnp.where(cid == 0, pw0, pw1)
        base = jnp.where(cid == 0, sid * pw0, split0 + sid * pw1)
        base2 = jnp.minimum(base, n_chunks - pwmax)
        off = base - base2
        pltpu.sync_copy(zero_hbm.at[pl.ds(r0, rows_per_sub)],
                        acc_sh.at[pl.ds(r0, rows_per_sub)])
        pltpu.sync_copy(src_hbm.at[pl.ds(base2, pwmax)], src_v)
        pltpu.sync_copy(dst_hbm.at[pl.ds(base2, pwmax)], dst_v)
        plsc.subcore_barrier()

        def gather_start(ki, b):
            @pl.when(ki < pe)
            def _():
                pltpu.async_copy(y_hbm.at[src_v.at[off + ki]], rows[b],
                                 gsem[b])

        def gather_wait(ki, b):
            @pl.when(ki < pe)
            def _():
                pltpu.make_async_copy(y_hbm.at[src_v.at[off + ki]], rows[b],
                                      gsem[b]).wait()

        def scat_start(ki, b):
            @pl.when(ki < pe)
            def _():
                pltpu.async_copy(rows[b], acc_sh.at[dst_v.at[off + ki]],
                                 ssem[b], add=True)

        def scat_wait(ki, b):
            @pl.when(ki < pe)
            def _():
                pltpu.make_async_copy(rows[b], acc_sh.at[dst_v.at[off + ki]],
                                      ssem[b]).wait()

        for b in range(look):
            gather_start(b, b)
        for b in range(nbuf):
            gather_wait(b, b)
            scat_start(b, b)
            b2 = (b + look) % nbuf
            if b >= look:
                scat_wait(b - look, b2)
            gather_start(look + b, b2)

        def body(g, carry):
            for b in range(nbuf):
                ki = g * nbuf + b
                gather_wait(ki, b)
                scat_start(ki, b)
                b2 = (b + look) % nbuf
                scat_wait(ki - look, b2)
                gather_start(ki + look, b2)
            return carry

        lax.fori_loop(1, ngroups, body, 0)
        for b in range(look, nbuf):
            scat_wait(pwmax - nbuf + b, b)
        plsc.subcore_barrier()
        pltpu.sync_copy(acc_sh.at[pl.ds(r0, rows_per_sub)],
                        out_hbm.at[cid, pl.ds(r0, rows_per_sub)])

    return k(y, src2d, dst2d, zeros)


def _tc_mid(agg, r1, b1, W2rel_p, W2root_p, b2_p, n):
    f = r1.shape[1]
    bn = 1000
    nacc = agg.shape[1]

    def body(agg_ref, r1_ref, b1_ref, wrel_ref, wroot_ref, b2_ref,
             y2_ref, r2_ref):
        h = jnp.maximum(
            agg_ref[0] + agg_ref[1] + b1_ref[...] + r1_ref[...], 0.0)
        y2_ref[...] = jnp.dot(h, wrel_ref[...],
                              preferred_element_type=jnp.float32)
        r2_ref[...] = jnp.dot(h, wroot_ref[...],
                              preferred_element_type=jnp.float32) + b2_ref[...]

    return pl.pallas_call(
        body,
        grid=(n // bn,),
        in_specs=[
            pl.BlockSpec((2, bn, f), lambda i: (0, i, 0)),
            pl.BlockSpec((bn, f), lambda i: (i, 0)),
            pl.BlockSpec((1, f), lambda i: (0, 0)),
            pl.BlockSpec((f, f), lambda i: (0, 0)),
            pl.BlockSpec((f, f), lambda i: (0, 0)),
            pl.BlockSpec((1, f), lambda i: (0, 0)),
        ],
        out_specs=[
            pl.BlockSpec((bn, f), lambda i: (i, 0)),
            pl.BlockSpec((bn, f), lambda i: (i, 0)),
        ],
        out_shape=[
            jax.ShapeDtypeStruct((n, f), jnp.float32),
            jax.ShapeDtypeStruct((n, f), jnp.float32),
        ],
    )(agg, r1, b1, W2rel_p, W2root_p, b2_p)


def _tc_logsoftmax(agg2, r2, n, c):
    f = r2.shape[1]
    bn = 1000

    def body(agg_ref, r2_ref, o_ref):
        logits = agg_ref[0] + agg_ref[1] + r2_ref[...]
        col = lax.broadcasted_iota(jnp.int32, logits.shape, 1)
        valid = col < c
        masked = jnp.where(valid, logits, -jnp.inf)
        m = jnp.max(masked, axis=1, keepdims=True)
        ex = jnp.where(valid, jnp.exp(logits - m), 0.0)
        s = jnp.sum(ex, axis=1, keepdims=True)
        o_ref[...] = logits - m - jnp.log(s)

    return pl.pallas_call(
        body,
        grid=(n // bn,),
        in_specs=[
            pl.BlockSpec((2, bn, f), lambda i: (0, i, 0)),
            pl.BlockSpec((bn, f), lambda i: (i, 0)),
        ],
        out_specs=pl.BlockSpec((bn, f), lambda i: (i, 0)),
        out_shape=jax.ShapeDtypeStruct((n, f), jnp.float32),
    )(agg2, r2)


def kernel(x, edge_index, W1_rel, b1, W1_root, W2_rel, b2, W2_root):
    n, d = x.shape
    e = edge_index.shape[1]
    h = W1_rel.shape[1]
    c = W2_rel.shape[1]
    nacc = -(-n // (_NSUB * 8)) * (_NSUB * 8)

    n_chunks = e // _CHUNK
    split0 = (n_chunks * 7 // 10) // 256 * 256
    src2d = edge_index[0].reshape(n_chunks, _CHUNK)
    dst2d = edge_index[1].reshape(n_chunks, _CHUNK)
    zeros = jnp.zeros((nacc, h), jnp.float32)

    W2rel_p = jnp.zeros((h, h), jnp.float32).at[:, :c].set(W2_rel)
    W2root_p = jnp.zeros((h, h), jnp.float32).at[:, :c].set(W2_root)
    b1_2d = b1.reshape(1, h)
    b2_p = jnp.zeros((1, h), jnp.float32).at[0, :c].set(b2)

    y1, r1 = _tc_dual_matmul(x, W1_rel, W1_root)
    agg1 = _segment_sum_sc(y1, src2d, dst2d, zeros, nacc, split0)
    y2, r2 = _tc_mid(agg1, r1, b1_2d, W2rel_p, W2root_p, b2_p, n)
    agg2 = _segment_sum_sc(y2, src2d, dst2d, zeros, nacc, split0)
    out = _tc_logsoftmax(agg2, r2, n, c)
    return out[:, :c]


# speedup vs baseline: 19.3164x; 1.0619x over previous
import functools

import jax
import jax.numpy as jnp
from jax import lax
from jax.experimental import pallas as pl
from jax.experimental.pallas import tpu as pltpu
from jax.experimental.pallas import tpu_sc as plsc

_NW = 32
_CHUNK = 125
_NSUB = 16


def _tc_dual_matmul(x, Wa, Wb):
    n, d = x.shape
    f = Wa.shape[1]
    bn = 1000
    assert n % bn == 0

    def body(x_ref, wa_ref, wb_ref, ya_ref, yb_ref):
        xv = x_ref[...]
        ya_ref[...] = jnp.dot(xv, wa_ref[...], preferred_element_type=jnp.float32)
        yb_ref[...] = jnp.dot(xv, wb_ref[...], preferred_element_type=jnp.float32)

    return pl.pallas_call(
        body,
        grid=(n // bn,),
        in_specs=[
            pl.BlockSpec((bn, d), lambda i: (i, 0)),
            pl.BlockSpec((d, f), lambda i: (0, 0)),
            pl.BlockSpec((d, f), lambda i: (0, 0)),
        ],
        out_specs=[
            pl.BlockSpec((bn, f), lambda i: (i, 0)),
            pl.BlockSpec((bn, f), lambda i: (i, 0)),
        ],
        out_shape=[
            jax.ShapeDtypeStruct((n, f), jnp.float32),
            jax.ShapeDtypeStruct((n, f), jnp.float32),
        ],
    )(x, Wa, Wb)


def _segment_sum_sc(y, src2d, dst2d, zeros, nacc, split0):
    n_chunks = src2d.shape[0]
    f = y.shape[1]
    rows_per_sub = nacc // _NSUB
    nbuf = 8
    look = nbuf // 2
    pw0 = split0 // _NSUB
    pw1 = (n_chunks - split0) // _NSUB
    pwmax = max(pw0, pw1)
    ngroups = pwmax // nbuf
    mesh = plsc.VectorSubcoreMesh(core_axis_name="c", subcore_axis_name="s")

    @functools.partial(
        pl.kernel,
        mesh=mesh,
        compiler_params=pltpu.CompilerParams(use_tc_tiling_on_sc=False),
        out_type=jax.ShapeDtypeStruct((2, nacc, f), jnp.float32),
        scratch_types=[
            pltpu.VMEM((pwmax, _CHUNK), jnp.int32),
            pltpu.VMEM((pwmax, _CHUNK), jnp.int32),
            pltpu.VMEM_SHARED((nacc, f), jnp.float32),
        ]
        + [pltpu.VMEM((_CHUNK, f), jnp.float32)] * nbuf
        + [pltpu.SemaphoreType.DMA] * (2 * nbuf),
    )
    def k(y_hbm, src_hbm, dst_hbm, zero_hbm, out_hbm, src_v, dst_v, acc_sh,
          *rest):
        rows = rest[:nbuf]
        gsem = rest[nbuf:2 * nbuf]
        ssem = rest[2 * nbuf:]
        cid = lax.axis_index("c")
        sid = lax.axis_index("s")
        r0 = sid * rows_per_sub
        pe = jnp.where(cid == 0, pw0, pw1)
        base = jnp.where(cid == 0, sid * pw0, split0 + sid * pw1)
        base2 = jnp.minimum(base, n_chunks - pwmax)
        off = base - base2
        cz = pltpu.async_copy(zero_hbm.at[pl.ds(r0, rows_per_sub)],
                              acc_sh.at[pl.ds(r0, rows_per_sub)], gsem[0])
        cs = pltpu.async_copy(src_hbm.at[pl.ds(base2, pwmax)], src_v, gsem[1])
        cd = pltpu.async_copy(dst_hbm.at[pl.ds(base2, pwmax)], dst_v, gsem[2])
        cz.wait()
        cs.wait()
        cd.wait()
        plsc.subcore_barrier()

        def gather_start(ki, b):
            @pl.when(ki < pe)
            def _():
                pltpu.async_copy(y_hbm.at[src_v.at[off + ki]], rows[b],
                                 gsem[b])

        def gather_wait(ki, b):
            @pl.when(ki < pe)
            def _():
                pltpu.make_async_copy(y_hbm.at[src_v.at[off + ki]], rows[b],
                                      gsem[b]).wait()

        def scat_start(ki, b):
            @pl.when(ki < pe)
            def _():
                pltpu.async_copy(rows[b], acc_sh.at[dst_v.at[off + ki]],
                                 ssem[b], add=True)

        def scat_wait(ki, b):
            @pl.when(ki < pe)
            def _():
                pltpu.make_async_copy(rows[b], acc_sh.at[dst_v.at[off + ki]],
                                      ssem[b]).wait()

        for b in range(look):
            gather_start(b, b)
        for b in range(nbuf):
            gather_wait(b, b)
            scat_start(b, b)
            b2 = (b + look) % nbuf
            if b >= look:
                scat_wait(b - look, b2)
            gather_start(look + b, b2)

        def body(g, carry):
            for b in range(nbuf):
                ki = g * nbuf + b
                gather_wait(ki, b)
                scat_start(ki, b)
                b2 = (b + look) % nbuf
                scat_wait(ki - look, b2)
                gather_start(ki + look, b2)
            return carry

        lax.fori_loop(1, ngroups, body, 0)
        for b in range(look, nbuf):
            scat_wait(pwmax - nbuf + b, b)
        plsc.subcore_barrier()
        pltpu.sync_copy(acc_sh.at[pl.ds(r0, rows_per_sub)],
                        out_hbm.at[cid, pl.ds(r0, rows_per_sub)])

    return k(y, src2d, dst2d, zeros)


def _tc_mid(agg, r1, b1, W2rel_p, W2root_p, b2_p, n):
    f = r1.shape[1]
    bn = 1000
    nacc = agg.shape[1]

    def body(agg_ref, r1_ref, b1_ref, wrel_ref, wroot_ref, b2_ref,
             y2_ref, r2_ref):
        h = jnp.maximum(
            agg_ref[0] + agg_ref[1] + b1_ref[...] + r1_ref[...], 0.0)
        y2_ref[...] = jnp.dot(h, wrel_ref[...],
                              preferred_element_type=jnp.float32)
        r2_ref[...] = jnp.dot(h, wroot_ref[...],
                              preferred_element_type=jnp.float32) + b2_ref[...]

    return pl.pallas_call(
        body,
        grid=(n // bn,),
        in_specs=[
            pl.BlockSpec((2, bn, f), lambda i: (0, i, 0)),
            pl.BlockSpec((bn, f), lambda i: (i, 0)),
            pl.BlockSpec((1, f), lambda i: (0, 0)),
            pl.BlockSpec((f, f), lambda i: (0, 0)),
            pl.BlockSpec((f, f), lambda i: (0, 0)),
            pl.BlockSpec((1, f), lambda i: (0, 0)),
        ],
        out_specs=[
            pl.BlockSpec((bn, f), lambda i: (i, 0)),
            pl.BlockSpec((bn, f), lambda i: (i, 0)),
        ],
        out_shape=[
            jax.ShapeDtypeStruct((n, f), jnp.float32),
            jax.ShapeDtypeStruct((n, f), jnp.float32),
        ],
    )(agg, r1, b1, W2rel_p, W2root_p, b2_p)


def _tc_logsoftmax(agg2, r2, n, c):
    f = r2.shape[1]
    bn = 1000

    def body(agg_ref, r2_ref, o_ref):
        logits = agg_ref[0] + agg_ref[1] + r2_ref[...]
        col = lax.broadcasted_iota(jnp.int32, logits.shape, 1)
        valid = col < c
        masked = jnp.where(valid, logits, -jnp.inf)
        m = jnp.max(masked, axis=1, keepdims=True)
        ex = jnp.where(valid, jnp.exp(logits - m), 0.0)
        s = jnp.sum(ex, axis=1, keepdims=True)
        o_ref[...] = logits - m - jnp.log(s)

    return pl.pallas_call(
        body,
        grid=(n // bn,),
        in_specs=[
            pl.BlockSpec((2, bn, f), lambda i: (0, i, 0)),
            pl.BlockSpec((bn, f), lambda i: (i, 0)),
        ],
        out_specs=pl.BlockSpec((bn, f), lambda i: (i, 0)),
        out_shape=jax.ShapeDtypeStruct((n, f), jnp.float32),
    )(agg2, r2)


def kernel(x, edge_index, W1_rel, b1, W1_root, W2_rel, b2, W2_root):
    n, d = x.shape
    e = edge_index.shape[1]
    h = W1_rel.shape[1]
    c = W2_rel.shape[1]
    nacc = -(-n // (_NSUB * 8)) * (_NSUB * 8)

    n_chunks = e // _CHUNK
    split0 = (n_chunks * 6 // 10) // 256 * 256
    src2d = edge_index[0].reshape(n_chunks, _CHUNK)
    dst2d = edge_index[1].reshape(n_chunks, _CHUNK)
    zeros = jnp.zeros((nacc, h), jnp.float32)

    W2rel_p = jnp.zeros((h, h), jnp.float32).at[:, :c].set(W2_rel)
    W2root_p = jnp.zeros((h, h), jnp.float32).at[:, :c].set(W2_root)
    b1_2d = b1.reshape(1, h)
    b2_p = jnp.zeros((1, h), jnp.float32).at[0, :c].set(b2)

    y1, r1 = _tc_dual_matmul(x, W1_rel, W1_root)
    agg1 = _segment_sum_sc(y1, src2d, dst2d, zeros, nacc, split0)
    y2, r2 = _tc_mid(agg1, r1, b1_2d, W2rel_p, W2root_p, b2_p, n)
    agg2 = _segment_sum_sc(y2, src2d, dst2d, zeros, nacc, split0)
    out = _tc_logsoftmax(agg2, r2, n, c)
    return out[:, :c]


# speedup vs baseline: 20.8044x; 1.0770x over previous
import functools

import jax
import jax.numpy as jnp
from jax import lax
from jax.experimental import pallas as pl
from jax.experimental.pallas import tpu as pltpu
from jax.experimental.pallas import tpu_sc as plsc

_NW = 32
_CHUNK = 125
_NSUB = 16


def _tc_dual_matmul(x, Wa, Wb):
    n, d = x.shape
    f = Wa.shape[1]
    bn = 1000
    assert n % bn == 0

    def body(x_ref, wa_ref, wb_ref, ya_ref, yb_ref):
        xv = x_ref[...]
        ya_ref[...] = jnp.dot(xv, wa_ref[...], preferred_element_type=jnp.float32)
        yb_ref[...] = jnp.dot(xv, wb_ref[...], preferred_element_type=jnp.float32)

    return pl.pallas_call(
        body,
        grid=(n // bn,),
        in_specs=[
            pl.BlockSpec((bn, d), lambda i: (i, 0)),
            pl.BlockSpec((d, f), lambda i: (0, 0)),
            pl.BlockSpec((d, f), lambda i: (0, 0)),
        ],
        out_specs=[
            pl.BlockSpec((bn, f), lambda i: (i, 0)),
            pl.BlockSpec((bn, f), lambda i: (i, 0)),
        ],
        out_shape=[
            jax.ShapeDtypeStruct((n, f), jnp.float32),
            jax.ShapeDtypeStruct((n, f), jnp.float32),
        ],
    )(x, Wa, Wb)


def _segment_sum_sc(y, edges2d, zeros, nacc, split0):
    n_chunks = edges2d.shape[0] // 2
    f = y.shape[1]
    rows_per_sub = nacc // _NSUB
    nbuf = 8
    look = nbuf // 2
    pw0 = split0 // _NSUB
    pw1 = (n_chunks - split0) // _NSUB
    pwmax = max(pw0, pw1)
    ngroups = pwmax // nbuf
    mesh = plsc.VectorSubcoreMesh(core_axis_name="c", subcore_axis_name="s")

    @functools.partial(
        pl.kernel,
        mesh=mesh,
        compiler_params=pltpu.CompilerParams(use_tc_tiling_on_sc=False),
        out_type=jax.ShapeDtypeStruct((2, nacc, f), jnp.float32),
        scratch_types=[
            pltpu.VMEM((pwmax, _CHUNK), jnp.int32),
            pltpu.VMEM((pwmax, _CHUNK), jnp.int32),
            pltpu.VMEM_SHARED((nacc, f), jnp.float32),
        ]
        + [pltpu.VMEM((_CHUNK, f), jnp.float32)] * nbuf
        + [pltpu.SemaphoreType.DMA] * (2 * nbuf),
    )
    def k(y_hbm, edges_hbm, zero_hbm, out_hbm, src_v, dst_v, acc_sh,
          *rest):
        rows = rest[:nbuf]
        gsem = rest[nbuf:2 * nbuf]
        ssem = rest[2 * nbuf:]
        cid = lax.axis_index("c")
        sid = lax.axis_index("s")
        r0 = sid * rows_per_sub
        pe = jnp.where(cid == 0, pw0, pw1)
        base = jnp.where(cid == 0, sid * pw0, split0 + sid * pw1)
        base2 = jnp.minimum(base, n_chunks - pwmax)
        off = base - base2
        cz = pltpu.async_copy(zero_hbm.at[pl.ds(r0, rows_per_sub)],
                              acc_sh.at[pl.ds(r0, rows_per_sub)], gsem[0])
        cs = pltpu.async_copy(edges_hbm.at[pl.ds(base2, pwmax)], src_v,
                              gsem[1])
        cd = pltpu.async_copy(edges_hbm.at[pl.ds(n_chunks + base2, pwmax)],
                              dst_v, gsem[2])
        cz.wait()
        cs.wait()
        cd.wait()
        plsc.subcore_barrier()

        def gather_start(ki, b):
            @pl.when(ki < pe)
            def _():
                pltpu.async_copy(y_hbm.at[src_v.at[off + ki]], rows[b],
                                 gsem[b])

        def gather_wait(ki, b):
            @pl.when(ki < pe)
            def _():
                pltpu.make_async_copy(y_hbm.at[src_v.at[off + ki]], rows[b],
                                      gsem[b]).wait()

        def scat_start(ki, b):
            @pl.when(ki < pe)
            def _():
                pltpu.async_copy(rows[b], acc_sh.at[dst_v.at[off + ki]],
                                 ssem[b], add=True)

        def scat_wait(ki, b):
            @pl.when(ki < pe)
            def _():
                pltpu.make_async_copy(rows[b], acc_sh.at[dst_v.at[off + ki]],
                                      ssem[b]).wait()

        for b in range(look):
            gather_start(b, b)
        for b in range(nbuf):
            gather_wait(b, b)
            scat_start(b, b)
            b2 = (b + look) % nbuf
            if b >= look:
                scat_wait(b - look, b2)
            gather_start(look + b, b2)

        def body(g, carry):
            for b in range(nbuf):
                ki = g * nbuf + b
                gather_wait(ki, b)
                scat_start(ki, b)
                b2 = (b + look) % nbuf
                scat_wait(ki - look, b2)
                gather_start(ki + look, b2)
            return carry

        lax.fori_loop(1, ngroups, body, 0)
        for b in range(look, nbuf):
            scat_wait(pwmax - nbuf + b, b)
        plsc.subcore_barrier()
        pltpu.sync_copy(acc_sh.at[pl.ds(r0, rows_per_sub)],
                        out_hbm.at[cid, pl.ds(r0, rows_per_sub)])

    return k(y, edges2d, zeros)


def _tc_mid(agg, r1, b1, W2rel_p, W2root_p, b2_p, n):
    f = r1.shape[1]
    bn = 1000
    nacc = agg.shape[1]

    def body(agg_ref, r1_ref, b1_ref, wrel_ref, wroot_ref, b2_ref,
             y2_ref, r2_ref):
        h = jnp.maximum(
            agg_ref[0] + agg_ref[1] + b1_ref[...] + r1_ref[...], 0.0)
        y2_ref[...] = jnp.dot(h, wrel_ref[...],
                              preferred_element_type=jnp.float32)
        r2_ref[...] = jnp.dot(h, wroot_ref[...],
                              preferred_element_type=jnp.float32) + b2_ref[...]

    return pl.pallas_call(
        body,
        grid=(n // bn,),
        in_specs=[
            pl.BlockSpec((2, bn, f), lambda i: (0, i, 0)),
            pl.BlockSpec((bn, f), lambda i: (i, 0)),
            pl.BlockSpec((1, f), lambda i: (0, 0)),
            pl.BlockSpec((f, f), lambda i: (0, 0)),
            pl.BlockSpec((f, f), lambda i: (0, 0)),
            pl.BlockSpec((1, f), lambda i: (0, 0)),
        ],
        out_specs=[
            pl.BlockSpec((bn, f), lambda i: (i, 0)),
            pl.BlockSpec((bn, f), lambda i: (i, 0)),
        ],
        out_shape=[
            jax.ShapeDtypeStruct((n, f), jnp.float32),
            jax.ShapeDtypeStruct((n, f), jnp.float32),
        ],
    )(agg, r1, b1, W2rel_p, W2root_p, b2_p)


def _tc_logsoftmax(agg2, r2, n, c):
    f = r2.shape[1]
    bn = 1000

    def body(agg_ref, r2_ref, o_ref):
        logits = agg_ref[0] + agg_ref[1] + r2_ref[...]
        col = lax.broadcasted_iota(jnp.int32, logits.shape, 1)
        valid = col < c
        masked = jnp.where(valid, logits, -jnp.inf)
        m = jnp.max(masked, axis=1, keepdims=True)
        ex = jnp.where(valid, jnp.exp(logits - m), 0.0)
        s = jnp.sum(ex, axis=1, keepdims=True)
        o_ref[...] = (logits - m - jnp.log(s))[:, :c]

    return pl.pallas_call(
        body,
        grid=(n // bn,),
        in_specs=[
            pl.BlockSpec((2, bn, f), lambda i: (0, i, 0)),
            pl.BlockSpec((bn, f), lambda i: (i, 0)),
        ],
        out_specs=pl.BlockSpec((bn, c), lambda i: (i, 0)),
        out_shape=jax.ShapeDtypeStruct((n, c), jnp.float32),
    )(agg2, r2)


def kernel(x, edge_index, W1_rel, b1, W1_root, W2_rel, b2, W2_root):
    n, d = x.shape
    e = edge_index.shape[1]
    h = W1_rel.shape[1]
    c = W2_rel.shape[1]
    nacc = -(-n // (_NSUB * 8)) * (_NSUB * 8)

    n_chunks = e // _CHUNK
    split0 = (n_chunks * 6 // 10) // 256 * 256
    edges2d = edge_index.reshape(2 * n_chunks, _CHUNK)
    zeros = jnp.zeros((nacc, h), jnp.float32)

    W2rel_p = jnp.zeros((h, h), jnp.float32).at[:, :c].set(W2_rel)
    W2root_p = jnp.zeros((h, h), jnp.float32).at[:, :c].set(W2_root)
    b1_2d = b1.reshape(1, h)
    b2_p = jnp.zeros((1, h), jnp.float32).at[0, :c].set(b2)

    y1, r1 = _tc_dual_matmul(x, W1_rel, W1_root)
    agg1 = _segment_sum_sc(y1, edges2d, zeros, nacc, split0)
    y2, r2 = _tc_mid(agg1, r1, b1_2d, W2rel_p, W2root_p, b2_p, n)
    agg2 = _segment_sum_sc(y2, edges2d, zeros, nacc, split0)
    return _tc_logsoftmax(agg2, r2, n, c)


# speedup vs baseline: 20.8087x; 1.0002x over previous
import functools

import jax
import jax.numpy as jnp
from jax import lax
from jax.experimental import pallas as pl
from jax.experimental.pallas import tpu as pltpu
from jax.experimental.pallas import tpu_sc as plsc

_NW = 32
_CHUNK = 125
_NSUB = 16


def _tc_dual_matmul(x, Wa, Wb):
    n, d = x.shape
    f = Wa.shape[1]
    bn = 1000
    assert n % bn == 0

    def body(x_ref, wa_ref, wb_ref, ya_ref, yb_ref):
        xv = x_ref[...]
        ya_ref[...] = jnp.dot(xv, wa_ref[...], preferred_element_type=jnp.float32)
        yb_ref[...] = jnp.dot(xv, wb_ref[...], preferred_element_type=jnp.float32)

    return pl.pallas_call(
        body,
        grid=(n // bn,),
        in_specs=[
            pl.BlockSpec((bn, d), lambda i: (i, 0)),
            pl.BlockSpec((d, f), lambda i: (0, 0)),
            pl.BlockSpec((d, f), lambda i: (0, 0)),
        ],
        out_specs=[
            pl.BlockSpec((bn, f), lambda i: (i, 0)),
            pl.BlockSpec((bn, f), lambda i: (i, 0)),
        ],
        out_shape=[
            jax.ShapeDtypeStruct((n, f), jnp.float32),
            jax.ShapeDtypeStruct((n, f), jnp.float32),
        ],
    )(x, Wa, Wb)


def _segment_sum_sc(y, edges2d, zeros, nacc, split0):
    n_chunks = edges2d.shape[0] // 2
    f = y.shape[1]
    rows_per_sub = nacc // _NSUB
    nbuf = 8
    look = nbuf // 2
    pw0 = split0 // _NSUB
    pw1 = (n_chunks - split0) // _NSUB
    pwmax = max(pw0, pw1)
    ngroups = pwmax // nbuf
    mesh = plsc.VectorSubcoreMesh(core_axis_name="c", subcore_axis_name="s")

    @functools.partial(
        pl.kernel,
        mesh=mesh,
        compiler_params=pltpu.CompilerParams(use_tc_tiling_on_sc=False),
        out_type=jax.ShapeDtypeStruct((2, nacc, f), jnp.float32),
        scratch_types=[
            pltpu.VMEM((pwmax, _CHUNK), jnp.int32),
            pltpu.VMEM((pwmax, _CHUNK), jnp.int32),
            pltpu.VMEM_SHARED((nacc, f), jnp.float32),
        ]
        + [pltpu.VMEM((_CHUNK, f), jnp.float32)] * nbuf
        + [pltpu.SemaphoreType.DMA] * (2 * nbuf),
    )
    def k(y_hbm, edges_hbm, zero_hbm, out_hbm, src_v, dst_v, acc_sh,
          *rest):
        rows = rest[:nbuf]
        gsem = rest[nbuf:2 * nbuf]
        ssem = rest[2 * nbuf:]
        cid = lax.axis_index("c")
        sid = lax.axis_index("s")
        r0 = sid * rows_per_sub
        pe = jnp.where(cid == 0, pw0, pw1)
        base = jnp.where(cid == 0, sid * pw0, split0 + sid * pw1)
        base2 = jnp.minimum(base, n_chunks - pwmax)
        off = base - base2
        cz = pltpu.async_copy(zero_hbm.at[pl.ds(r0, rows_per_sub)],
                              acc_sh.at[pl.ds(r0, rows_per_sub)], gsem[0])
        cs = pltpu.async_copy(edges_hbm.at[pl.ds(base2, pwmax)], src_v,
                              gsem[1])
        cd = pltpu.async_copy(edges_hbm.at[pl.ds(n_chunks + base2, pwmax)],
                              dst_v, gsem[2])
        cz.wait()
        cs.wait()
        cd.wait()
        plsc.subcore_barrier()
        y_r = y_hbm

        def gather_start(ki, b):
            @pl.when(ki < pe)
            def _():
                pltpu.async_copy(y_r.at[src_v.at[off + ki]], rows[b],
                                 gsem[b])

        def gather_wait(ki, b):
            @pl.when(ki < pe)
            def _():
                pltpu.make_async_copy(y_r.at[src_v.at[off + ki]], rows[b],
                                      gsem[b]).wait()

        def scat_start(ki, b):
            @pl.when(ki < pe)
            def _():
                pltpu.async_copy(rows[b], acc_sh.at[dst_v.at[off + ki]],
                                 ssem[b], add=True)

        def scat_wait(ki, b):
            @pl.when(ki < pe)
            def _():
                pltpu.make_async_copy(rows[b], acc_sh.at[dst_v.at[off + ki]],
                                      ssem[b]).wait()

        for b in range(look):
            gather_start(b, b)
        for b in range(nbuf):
            gather_wait(b, b)
            scat_start(b, b)
            b2 = (b + look) % nbuf
            if b >= look:
                scat_wait(b - look, b2)
            gather_start(look + b, b2)

        def body(g, carry):
            for b in range(nbuf):
                ki = g * nbuf + b
                gather_wait(ki, b)
                scat_start(ki, b)
                b2 = (b + look) % nbuf
                scat_wait(ki - look, b2)
                gather_start(ki + look, b2)
            return carry

        lax.fori_loop(1, ngroups, body, 0)
        for b in range(look, nbuf):
            scat_wait(pwmax - nbuf + b, b)
        plsc.subcore_barrier()
        pltpu.sync_copy(acc_sh.at[pl.ds(r0, rows_per_sub)],
                        out_hbm.at[cid, pl.ds(r0, rows_per_sub)])

    return k(y, edges2d, zeros)


def _tc_mid(agg, r1, b1, W2rel_p, W2root_p, b2_p, n):
    f = agg.shape[2]
    bn = 1000

    def body(agg_ref, r1_ref, b1_ref, wrel_ref, wroot_ref, b2_ref,
             y2_ref, r2_ref):
        h = jnp.maximum(
            agg_ref[0] + agg_ref[1] + b1_ref[...] + r1_ref[...], 0.0)
        y2_ref[...] = jnp.dot(h, wrel_ref[...],
                              preferred_element_type=jnp.float32)
        r2_ref[...] = jnp.dot(h, wroot_ref[...],
                              preferred_element_type=jnp.float32) + b2_ref[...]

    return pl.pallas_call(
        body,
        grid=(n // bn,),
        in_specs=[
            pl.BlockSpec((2, bn, f), lambda i: (0, i, 0)),
            pl.BlockSpec((bn, f), lambda i: (i, 0)),
            pl.BlockSpec((1, f), lambda i: (0, 0)),
            pl.BlockSpec((f, f), lambda i: (0, 0)),
            pl.BlockSpec((f, f), lambda i: (0, 0)),
            pl.BlockSpec((1, f), lambda i: (0, 0)),
        ],
        out_specs=[
            pl.BlockSpec((bn, f), lambda i: (i, 0)),
            pl.BlockSpec((bn, f), lambda i: (i, 0)),
        ],
        out_shape=[
            jax.ShapeDtypeStruct((n, f), jnp.float32),
            jax.ShapeDtypeStruct((n, f), jnp.float32),
        ],
    )(agg, r1, b1, W2rel_p, W2root_p, b2_p)


def _tc_logsoftmax(agg2, r2, n, c):
    f = r2.shape[1]
    bn = 1000

    def body(agg_ref, r2_ref, o_ref):
        logits = agg_ref[0] + agg_ref[1] + r2_ref[...]
        col = lax.broadcasted_iota(jnp.int32, logits.shape, 1)
        valid = col < c
        masked = jnp.where(valid, logits, -jnp.inf)
        m = jnp.max(masked, axis=1, keepdims=True)
        ex = jnp.where(valid, jnp.exp(logits - m), 0.0)
        s = jnp.sum(ex, axis=1, keepdims=True)
        o_ref[...] = (logits - m - jnp.log(s))[:, :c]

    return pl.pallas_call(
        body,
        grid=(n // bn,),
        in_specs=[
            pl.BlockSpec((2, bn, f), lambda i: (0, i, 0)),
            pl.BlockSpec((bn, f), lambda i: (i, 0)),
        ],
        out_specs=pl.BlockSpec((bn, c), lambda i: (i, 0)),
        out_shape=jax.ShapeDtypeStruct((n, c), jnp.float32),
    )(agg2, r2)


def kernel(x, edge_index, W1_rel, b1, W1_root, W2_rel, b2, W2_root):
    n, d = x.shape
    e = edge_index.shape[1]
    h = W1_rel.shape[1]
    c = W2_rel.shape[1]
    nacc = -(-n // (_NSUB * 8)) * (_NSUB * 8)

    n_chunks = e // _CHUNK
    split0 = (n_chunks * 6 // 10) // 256 * 256
    edges2d = edge_index.reshape(2 * n_chunks, _CHUNK)
    zeros = jnp.zeros((nacc, h), jnp.float32)

    W2rel_p = jnp.zeros((h, h), jnp.float32).at[:, :c].set(W2_rel)
    W2root_p = jnp.zeros((h, h), jnp.float32).at[:, :c].set(W2_root)
    b1_2d = b1.reshape(1, h)
    b2_p = jnp.zeros((1, h), jnp.float32).at[0, :c].set(b2)

    y1, r1 = _tc_dual_matmul(x, W1_rel, W1_root)
    agg1 = _segment_sum_sc(y1, edges2d, zeros, nacc, split0)
    y2, r2 = _tc_mid(agg1, r1, b1_2d, W2rel_p, W2root_p, b2_p, n)
    agg2 = _segment_sum_sc(y2, edges2d, zeros, nacc, split0)
    return _tc_logsoftmax(agg2, r2, n, c)


# speedup vs baseline: 21.8819x; 1.0516x over previous
import functools

import jax
import jax.numpy as jnp
from jax import lax
from jax.experimental import pallas as pl
from jax.experimental.pallas import tpu as pltpu
from jax.experimental.pallas import tpu_sc as plsc

_NW = 32
_CHUNK = 125
_NSUB = 16


def _tc_dual_matmul(x, Wa, Wb):
    n, d = x.shape
    f = Wa.shape[1]
    bn = 1000
    assert n % bn == 0

    def body(x_ref, wa_ref, wb_ref, ya_ref, yb_ref):
        xv = x_ref[...]
        ya_ref[...] = jnp.dot(xv, wa_ref[...], preferred_element_type=jnp.float32)
        yb_ref[...] = jnp.dot(xv, wb_ref[...], preferred_element_type=jnp.float32)

    return pl.pallas_call(
        body,
        grid=(n // bn,),
        in_specs=[
            pl.BlockSpec((bn, d), lambda i: (i, 0)),
            pl.BlockSpec((d, f), lambda i: (0, 0)),
            pl.BlockSpec((d, f), lambda i: (0, 0)),
        ],
        out_specs=[
            pl.BlockSpec((bn, f), lambda i: (i, 0)),
            pl.BlockSpec((bn, f), lambda i: (i, 0)),
        ],
        out_shape=[
            jax.ShapeDtypeStruct((n, f), jnp.float32),
            jax.ShapeDtypeStruct((n, f), jnp.float32),
        ],
    )(x, Wa, Wb)


def _segment_sum_sc(y, edges2d, zeros, nacc, split0):
    n_chunks = edges2d.shape[0] // 2
    f = y.shape[1]
    rows_per_sub = nacc // _NSUB
    nbuf = 8
    look = nbuf // 2
    pw0 = split0 // _NSUB
    pw1 = (n_chunks - split0) // _NSUB
    pwmax = max(pw0, pw1)
    ngroups = pwmax // nbuf
    mesh = plsc.VectorSubcoreMesh(core_axis_name="c", subcore_axis_name="s")

    @functools.partial(
        pl.kernel,
        mesh=mesh,
        compiler_params=pltpu.CompilerParams(use_tc_tiling_on_sc=False),
        out_type=jax.ShapeDtypeStruct((2, nacc, f), jnp.float32),
        scratch_types=[
            pltpu.VMEM((pwmax, _CHUNK), jnp.int32),
            pltpu.VMEM((pwmax, _CHUNK), jnp.int32),
            pltpu.VMEM_SHARED((nacc, f), jnp.float32),
        ]
        + [pltpu.VMEM((_CHUNK, f), jnp.float32)] * nbuf
        + [pltpu.SemaphoreType.DMA] * (2 * nbuf),
    )
    def k(y_hbm, edges_hbm, zero_hbm, out_hbm, src_v, dst_v, acc_sh,
          *rest):
        rows = rest[:nbuf]
        gsem = rest[nbuf:2 * nbuf]
        ssem = rest[2 * nbuf:]
        cid = lax.axis_index("c")
        sid = lax.axis_index("s")
        r0 = sid * rows_per_sub
        pe = jnp.where(cid == 0, pw0, pw1)
        base = jnp.where(cid == 0, sid * pw0, split0 + sid * pw1)
        base2 = jnp.minimum(base, n_chunks - pwmax)
        off = base - base2
        cz = pltpu.async_copy(zero_hbm.at[pl.ds(r0, rows_per_sub)],
                              acc_sh.at[pl.ds(r0, rows_per_sub)], gsem[0])
        cs = pltpu.async_copy(edges_hbm.at[pl.ds(base2, pwmax)], src_v,
                              gsem[1])
        cd = pltpu.async_copy(edges_hbm.at[pl.ds(n_chunks + base2, pwmax)],
                              dst_v, gsem[2])
        cz.wait()
        cs.wait()
        cd.wait()
        plsc.subcore_barrier()
        y_r = y_hbm

        def gather_start(ki, b):
            @pl.when(ki < pe)
            def _():
                pltpu.async_copy(y_r.at[src_v.at[off + ki]], rows[b],
                                 gsem[b])

        def gather_wait(ki, b):
            @pl.when(ki < pe)
            def _():
                pltpu.make_async_copy(y_r.at[src_v.at[off + ki]], rows[b],
                                      gsem[b]).wait()

        def scat_start(ki, b):
            @pl.when(ki < pe)
            def _():
                pltpu.async_copy(rows[b], acc_sh.at[dst_v.at[off + ki]],
                                 ssem[b], add=True)

        def scat_wait(ki, b):
            @pl.when(ki < pe)
            def _():
                pltpu.make_async_copy(rows[b], acc_sh.at[dst_v.at[off + ki]],
                                      ssem[b]).wait()

        for b in range(look):
            gather_start(b, b)
        for b in range(nbuf):
            gather_wait(b, b)
            scat_start(b, b)
            b2 = (b + look) % nbuf
            if b >= look:
                scat_wait(b - look, b2)
            gather_start(look + b, b2)

        def body(g, carry):
            for b in range(nbuf):
                ki = g * nbuf + b
                gather_wait(ki, b)
                scat_start(ki, b)
                b2 = (b + look) % nbuf
                scat_wait(ki - look, b2)
                gather_start(ki + look, b2)
            return carry

        lax.fori_loop(1, ngroups, body, 0)
        for b in range(look, nbuf):
            scat_wait(pwmax - nbuf + b, b)
        plsc.subcore_barrier()
        pltpu.sync_copy(acc_sh.at[pl.ds(r0, rows_per_sub)],
                        out_hbm.at[cid, pl.ds(r0, rows_per_sub)])

    return k(y, edges2d, zeros)


def _tc_mid(agg, r1, b1, W2rel_p, W2root_p, b2_p, n):
    f = agg.shape[2]
    bn = 1000

    def body(agg_ref, r1_ref, b1_ref, wrel_ref, wroot_ref, b2_ref,
             y2_ref, r2_ref):
        h = jnp.maximum(
            agg_ref[0] + agg_ref[1] + b1_ref[...] + r1_ref[...], 0.0)
        y2_ref[...] = jnp.dot(h, wrel_ref[...],
                              preferred_element_type=jnp.float32)
        r2_ref[...] = jnp.dot(h, wroot_ref[...],
                              preferred_element_type=jnp.float32) + b2_ref[...]

    return pl.pallas_call(
        body,
        grid=(n // bn,),
        in_specs=[
            pl.BlockSpec((2, bn, f), lambda i: (0, i, 0)),
            pl.BlockSpec((bn, f), lambda i: (i, 0)),
            pl.BlockSpec((1, f), lambda i: (0, 0)),
            pl.BlockSpec((f, f), lambda i: (0, 0)),
            pl.BlockSpec((f, f), lambda i: (0, 0)),
            pl.BlockSpec((1, f), lambda i: (0, 0)),
        ],
        out_specs=[
            pl.BlockSpec((bn, f), lambda i: (i, 0)),
            pl.BlockSpec((bn, f), lambda i: (i, 0)),
        ],
        out_shape=[
            jax.ShapeDtypeStruct((n, f), jnp.float32),
            jax.ShapeDtypeStruct((n, f), jnp.float32),
        ],
    )(agg, r1, b1, W2rel_p, W2root_p, b2_p)


def _tc_logsoftmax(agg2, r2, n, c):
    f = r2.shape[1]
    bn = 1000

    def body(agg_ref, r2_ref, o_ref):
        logits = agg_ref[0] + agg_ref[1] + r2_ref[...]
        col = lax.broadcasted_iota(jnp.int32, logits.shape, 1)
        valid = col < c
        masked = jnp.where(valid, logits, -jnp.inf)
        m = jnp.max(masked, axis=1, keepdims=True)
        ex = jnp.where(valid, jnp.exp(logits - m), 0.0)
        s = jnp.sum(ex, axis=1, keepdims=True)
        o_ref[...] = (logits - m - jnp.log(s))[:, :c]

    return pl.pallas_call(
        body,
        grid=(n // bn,),
        in_specs=[
            pl.BlockSpec((2, bn, f), lambda i: (0, i, 0)),
            pl.BlockSpec((bn, f), lambda i: (i, 0)),
        ],
        out_specs=pl.BlockSpec((bn, c), lambda i: (i, 0)),
        out_shape=jax.ShapeDtypeStruct((n, c), jnp.float32),
    )(agg2, r2)


def kernel(x, edge_index, W1_rel, b1, W1_root, W2_rel, b2, W2_root):
    n, d = x.shape
    e = edge_index.shape[1]
    h = W1_rel.shape[1]
    c = W2_rel.shape[1]
    nacc = -(-n // (_NSUB * 8)) * (_NSUB * 8)

    n_chunks = e // _CHUNK
    split0 = (n_chunks * 55 // 100) // 256 * 256
    edges2d = edge_index.reshape(2 * n_chunks, _CHUNK)
    zeros = jnp.zeros((nacc, h), jnp.float32)

    W2rel_p = jnp.zeros((h, h), jnp.float32).at[:, :c].set(W2_rel)
    W2root_p = jnp.zeros((h, h), jnp.float32).at[:, :c].set(W2_root)
    b1_2d = b1.reshape(1, h)
    b2_p = jnp.zeros((1, h), jnp.float32).at[0, :c].set(b2)

    y1, r1 = _tc_dual_matmul(x, W1_rel, W1_root)
    agg1 = _segment_sum_sc(y1, edges2d, zeros, nacc, split0)
    y2, r2 = _tc_mid(agg1, r1, b1_2d, W2rel_p, W2root_p, b2_p, n)
    agg2 = _segment_sum_sc(y2, edges2d, zeros, nacc, split0)
    return _tc_logsoftmax(agg2, r2, n, c)
